# baseline (speedup 1.0000x reference)
import functools

import jax
import jax.numpy as jnp
from jax import lax
from jax.experimental import pallas as pl
from jax.experimental.pallas import tpu as pltpu
from jax.experimental.pallas import tpu_sc as plsc

_NC = 2
_NS = 16
_NW = _NC * _NS
_LANES = 16
_BATCH = 16384
_DIM = 128
_CHUNK = 128

_SPLIT = 8192
_REST = _BATCH - _SPLIT
_BPW_A = _SPLIT // _NW
_BPW_B = _REST // _NW
_NCH_A = _BPW_A // _CHUNK
_NCH_B = _BPW_B // _CHUNK
_IND_ROWS_A = _SPLIT // _CHUNK
_VECS = _DIM // _LANES

_TC_BLK = 512


def _sc_gather_body(ind_hbm, b_hbm, out_hbm, idx_v, rows_v, gsem, ssem):
    c = lax.axis_index("c")
    s = lax.axis_index("s")
    wid = c * _NS + s
    base = wid * _BPW_A

    pltpu.sync_copy(ind_hbm.at[pl.ds(wid * _NCH_A, _NCH_A)], idx_v)

    gathers = [None] * _NCH_A
    stores = [None] * _NCH_A
    gathers[0] = pltpu.async_copy(
        b_hbm.at[idx_v.at[0]], rows_v.at[0], gsem.at[0])
    for ch in range(_NCH_A):
        buf = ch % 2
        gathers[ch].wait()
        stores[ch] = pltpu.async_copy(
            rows_v.at[buf], out_hbm.at[pl.ds(base + ch * _CHUNK, _CHUNK)],
            ssem.at[buf])
        if ch + 1 < _NCH_A:
            if ch >= 1:
                stores[ch - 1].wait()
            gathers[ch + 1] = pltpu.async_copy(
                b_hbm.at[idx_v.at[ch + 1]], rows_v.at[1 - buf],
                gsem.at[1 - buf])
    for ch in range(max(0, _NCH_A - 2), _NCH_A):
        stores[ch].wait()


def _sc_reduce_body(ind_hbm, eeg_hbm, ir_hbm, b_hbm, out_hbm,
                    idx_v, rows_v, eeg_v, ir_v, acc_v, sems):
    c = lax.axis_index("c")
    s = lax.axis_index("s")
    wid = c * _NS + s
    base = _SPLIT + wid * _BPW_B

    pltpu.sync_copy(
        ind_hbm.at[pl.ds(_IND_ROWS_A + wid * _NCH_B, _NCH_B)], idx_v)

    def fire(ch):
        buf = ch % 2
        row0 = base + ch * _CHUNK
        return (
            pltpu.async_copy(b_hbm.at[idx_v.at[ch]], rows_v.at[buf],
                             sems.at[buf, 0]),
            pltpu.async_copy(eeg_hbm.at[pl.ds(row0, _CHUNK)], eeg_v.at[buf],
                             sems.at[buf, 1]),
            pltpu.async_copy(ir_hbm.at[pl.ds(row0, _CHUNK)], ir_v.at[buf],
                             sems.at[buf, 2]),
        )

    zero = jnp.zeros((_LANES,), jnp.float32)
    acc_e = zero
    acc_i = zero

    inflight = fire(0)
    for ch in range(_NCH_B):
        buf = ch % 2
        for cp in inflight:
            cp.wait()
        if ch + 1 < _NCH_B:
            inflight = fire(ch + 1)

        @plsc.parallel_loop(0, _CHUNK, unroll=8, carry=(acc_e, acc_i))
        def _row(r, carry):
            a_e, a_i = carry
            for j in range(_VECS):
                col = j * _LANES
                b = rows_v[buf, r, pl.ds(col, _LANES)]
                e = eeg_v[buf, r, pl.ds(col, _LANES)]
                i = ir_v[buf, r, pl.ds(col, _LANES)]
                de = b - e
                di = b - i
                a_e = a_e + de * de
                a_i = a_i + di * di
            return a_e, a_i

        acc_e, acc_i = _row

    acc_v[...] = acc_e + acc_i
    pltpu.sync_copy(acc_v, out_hbm.at[wid])


def _tc_reduce_kernel(rows_ref, eeg_ref, ir_ref, acc_ref):
    i = pl.program_id(0)
    b = rows_ref[...]
    de = b - eeg_ref[...]
    di = b - ir_ref[...]
    part = de * de + di * di

    @pl.when(i == 0)
    def _():
        acc_ref[...] = part

    @pl.when(i != 0)
    def _():
        acc_ref[...] += part


@jax.jit
def _quant_loss(ind2, eeg, ir, b):
    mesh = plsc.VectorSubcoreMesh(
        core_axis_name="c", subcore_axis_name="s",
        num_cores=_NC, num_subcores=_NS)

    rows_a = pl.kernel(
        _sc_gather_body,
        out_type=jax.ShapeDtypeStruct((_SPLIT, _DIM), jnp.float32),
        mesh=mesh,
        scratch_types=[
            pltpu.VMEM((_NCH_A, _CHUNK), jnp.int32),
            pltpu.VMEM((2, _CHUNK, _DIM), jnp.float32),
            pltpu.SemaphoreType.DMA((2,)),
            pltpu.SemaphoreType.DMA((2,)),
        ],
    )(ind2, b)

    partials_b = pl.kernel(
        _sc_reduce_body,
        out_type=jax.ShapeDtypeStruct((_NW, _LANES), jnp.float32),
        mesh=mesh,
        scratch_types=[
            pltpu.VMEM((_NCH_B, _CHUNK), jnp.int32),
            pltpu.VMEM((2, _CHUNK, _DIM), jnp.float32),
            pltpu.VMEM((2, _CHUNK, _DIM), jnp.float32),
            pltpu.VMEM((2, _CHUNK, _DIM), jnp.float32),
            pltpu.VMEM((_LANES,), jnp.float32),
            pltpu.SemaphoreType.DMA((2, 3)),
        ],
    )(ind2, eeg, ir, b)

    acc_a = pl.pallas_call(
        _tc_reduce_kernel,
        grid=(_SPLIT // _TC_BLK,),
        in_specs=[
            pl.BlockSpec((_TC_BLK, _DIM), lambda i: (i, 0)),
            pl.BlockSpec((_TC_BLK, _DIM), lambda i: (i, 0)),
            pl.BlockSpec((_TC_BLK, _DIM), lambda i: (i, 0)),
        ],
        out_specs=pl.BlockSpec((_TC_BLK, _DIM), lambda i: (0, 0)),
        out_shape=jax.ShapeDtypeStruct((_TC_BLK, _DIM), jnp.float32),
    )(rows_a, eeg, ir)

    return jnp.sum(acc_a) + jnp.sum(partials_b)


def kernel(eeg, ir, ind, B, un_eeg, un_ir, device):
    ind2 = ind.astype(jnp.int32).reshape(_BATCH // _CHUNK, _CHUNK)
    return _quant_loss(ind2, eeg, ir, B)


# speedup vs baseline: 1.0380x; 1.0380x over previous
import functools

import jax
import jax.numpy as jnp
from jax import lax
from jax.experimental import pallas as pl
from jax.experimental.pallas import tpu as pltpu
from jax.experimental.pallas import tpu_sc as plsc

_NC = 2
_NS = 16
_NW = _NC * _NS
_LANES = 16
_BATCH = 16384
_DIM = 128
_CHUNK = 128

_SPLIT = 8192
_REST = _BATCH - _SPLIT
_BPW_A = _SPLIT // _NW
_BPW_B = _REST // _NW
_NCH_A = _BPW_A // _CHUNK
_NCH_B = _BPW_B // _CHUNK
_IND_ROWS_A = _SPLIT // _CHUNK
_VECS = _DIM // _LANES

_TC_BLK = 1024


def _sc_gather_body(ind_hbm, b_hbm, out_hbm, idx_v, rows_v, gsem, ssem):
    c = lax.axis_index("c")
    s = lax.axis_index("s")
    wid = c * _NS + s
    base = wid * _BPW_A

    pltpu.sync_copy(ind_hbm.at[pl.ds(wid * _NCH_A, _NCH_A)], idx_v)

    gathers = [None] * _NCH_A
    stores = [None] * _NCH_A
    gathers[0] = pltpu.async_copy(
        b_hbm.at[idx_v.at[0]], rows_v.at[0], gsem.at[0])
    for ch in range(_NCH_A):
        buf = ch % 2
        gathers[ch].wait()
        stores[ch] = pltpu.async_copy(
            rows_v.at[buf], out_hbm.at[pl.ds(base + ch * _CHUNK, _CHUNK)],
            ssem.at[buf])
        if ch + 1 < _NCH_A:
            if ch >= 1:
                stores[ch - 1].wait()
            gathers[ch + 1] = pltpu.async_copy(
                b_hbm.at[idx_v.at[ch + 1]], rows_v.at[1 - buf],
                gsem.at[1 - buf])
    for ch in range(max(0, _NCH_A - 2), _NCH_A):
        stores[ch].wait()


def _sc_reduce_body(ind_hbm, eeg_hbm, ir_hbm, b_hbm, out_hbm,
                    idx_v, rows_v, eeg_v, ir_v, acc_v, sems):
    c = lax.axis_index("c")
    s = lax.axis_index("s")
    wid = c * _NS + s
    base = _SPLIT + wid * _BPW_B

    pltpu.sync_copy(
        ind_hbm.at[pl.ds(_IND_ROWS_A + wid * _NCH_B, _NCH_B)], idx_v)

    def fire(ch):
        buf = ch % 2
        row0 = base + ch * _CHUNK
        return (
            pltpu.async_copy(b_hbm.at[idx_v.at[ch]], rows_v.at[buf],
                             sems.at[buf, 0]),
            pltpu.async_copy(eeg_hbm.at[pl.ds(row0, _CHUNK)], eeg_v.at[buf],
                             sems.at[buf, 1]),
            pltpu.async_copy(ir_hbm.at[pl.ds(row0, _CHUNK)], ir_v.at[buf],
                             sems.at[buf, 2]),
        )

    zero = jnp.zeros((_LANES,), jnp.float32)
    acc_e = zero
    acc_i = zero

    inflight = fire(0)
    for ch in range(_NCH_B):
        buf = ch % 2
        for cp in inflight:
            cp.wait()
        if ch + 1 < _NCH_B:
            inflight = fire(ch + 1)

        @plsc.parallel_loop(0, _CHUNK, unroll=8, carry=(acc_e, acc_i))
        def _row(r, carry):
            a_e, a_i = carry
            for j in range(_VECS):
                col = j * _LANES
                b = rows_v[buf, r, pl.ds(col, _LANES)]
                e = eeg_v[buf, r, pl.ds(col, _LANES)]
                i = ir_v[buf, r, pl.ds(col, _LANES)]
                de = b - e
                di = b - i
                a_e = a_e + de * de
                a_i = a_i + di * di
            return a_e, a_i

        acc_e, acc_i = _row

    acc_v[...] = acc_e + acc_i
    pltpu.sync_copy(acc_v, out_hbm.at[wid])


def _tc_reduce_kernel(rows_ref, eeg_ref, ir_ref, out_ref, acc_ref):
    i = pl.program_id(0)
    b = rows_ref[...]
    de = b - eeg_ref[...]
    di = b - ir_ref[...]
    part = de * de + di * di
    psum = jnp.sum(part.reshape(_TC_BLK // 8, 8, _DIM), axis=0)

    @pl.when(i == 0)
    def _():
        acc_ref[...] = psum

    @pl.when(i != 0)
    def _():
        acc_ref[...] += psum

    @pl.when(i == pl.num_programs(0) - 1)
    def _():
        out_ref[...] = acc_ref[...]


@jax.jit
def _quant_loss(ind2, eeg, ir, b):
    mesh = plsc.VectorSubcoreMesh(
        core_axis_name="c", subcore_axis_name="s",
        num_cores=_NC, num_subcores=_NS)

    rows_a = pl.kernel(
        _sc_gather_body,
        out_type=jax.ShapeDtypeStruct((_SPLIT, _DIM), jnp.float32),
        mesh=mesh,
        scratch_types=[
            pltpu.VMEM((_NCH_A, _CHUNK), jnp.int32),
            pltpu.VMEM((2, _CHUNK, _DIM), jnp.float32),
            pltpu.SemaphoreType.DMA((2,)),
            pltpu.SemaphoreType.DMA((2,)),
        ],
    )(ind2, b)

    partials_b = pl.kernel(
        _sc_reduce_body,
        out_type=jax.ShapeDtypeStruct((_NW, _LANES), jnp.float32),
        mesh=mesh,
        scratch_types=[
            pltpu.VMEM((_NCH_B, _CHUNK), jnp.int32),
            pltpu.VMEM((2, _CHUNK, _DIM), jnp.float32),
            pltpu.VMEM((2, _CHUNK, _DIM), jnp.float32),
            pltpu.VMEM((2, _CHUNK, _DIM), jnp.float32),
            pltpu.VMEM((_LANES,), jnp.float32),
            pltpu.SemaphoreType.DMA((2, 3)),
        ],
    )(ind2, eeg, ir, b)

    acc_a = pl.pallas_call(
        _tc_reduce_kernel,
        grid=(_SPLIT // _TC_BLK,),
        in_specs=[
            pl.BlockSpec((_TC_BLK, _DIM), lambda i: (i, 0)),
            pl.BlockSpec((_TC_BLK, _DIM), lambda i: (i, 0)),
            pl.BlockSpec((_TC_BLK, _DIM), lambda i: (i, 0)),
        ],
        out_specs=pl.BlockSpec((8, _DIM), lambda i: (0, 0)),
        out_shape=jax.ShapeDtypeStruct((8, _DIM), jnp.float32),
        scratch_shapes=[pltpu.VMEM((8, _DIM), jnp.float32)],
    )(rows_a, eeg, ir)

    return jnp.sum(acc_a) + jnp.sum(partials_b)


def kernel(eeg, ir, ind, B, un_eeg, un_ir, device):
    ind2 = ind.astype(jnp.int32).reshape(_BATCH // _CHUNK, _CHUNK)
    return _quant_loss(ind2, eeg, ir, B)


# speedup vs baseline: 1.2757x; 1.2290x over previous
import functools

import jax
import jax.numpy as jnp
from jax import lax
from jax.experimental import pallas as pl
from jax.experimental.pallas import tpu as pltpu
from jax.experimental.pallas import tpu_sc as plsc

_NC = 2
_NS = 16
_NW = _NC * _NS
_LANES = 16
_BATCH = 16384
_DIM = 128
_BPW = _BATCH // _NW
_CHUNK = 64
_NCHUNK = _BPW // _CHUNK
_NBUF = 3
_VECS = _DIM // _LANES


def _sc_body(ind_hbm, eeg_hbm, ir_hbm, b_hbm, out_hbm,
             idx_v, rows_v, eeg_v, ir_v, acc_v, sems):
    c = lax.axis_index("c")
    s = lax.axis_index("s")
    wid = c * _NS + s
    base = wid * _BPW

    pltpu.sync_copy(ind_hbm.at[pl.ds(wid * _NCHUNK, _NCHUNK)], idx_v)

    def fire(ch):
        buf = ch % _NBUF
        row0 = base + ch * _CHUNK
        return (
            pltpu.async_copy(b_hbm.at[idx_v.at[ch]], rows_v.at[buf],
                             sems.at[buf, 0]),
            pltpu.async_copy(eeg_hbm.at[pl.ds(row0, _CHUNK)], eeg_v.at[buf],
                             sems.at[buf, 1]),
            pltpu.async_copy(ir_hbm.at[pl.ds(row0, _CHUNK)], ir_v.at[buf],
                             sems.at[buf, 2]),
        )

    zero = jnp.zeros((_LANES,), jnp.float32)
    acc_e = zero
    acc_i = zero

    inflight = [fire(0), fire(1)]
    for ch in range(_NCHUNK):
        buf = ch % _NBUF
        for cp in inflight[0]:
            cp.wait()
        inflight = inflight[1:]
        if ch + 2 < _NCHUNK:
            inflight.append(fire(ch + 2))

        @plsc.parallel_loop(0, _CHUNK, unroll=4, carry=(acc_e, acc_i))
        def _row(r, carry):
            a_e, a_i = carry
            for j in range(_VECS):
                col = j * _LANES
                b = rows_v[buf, r, pl.ds(col, _LANES)]
                e = eeg_v[buf, r, pl.ds(col, _LANES)]
                i = ir_v[buf, r, pl.ds(col, _LANES)]
                de = b - e
                di = b - i
                a_e = a_e + de * de
                a_i = a_i + di * di
            return a_e, a_i

        acc_e, acc_i = _row

    acc_v[...] = acc_e + acc_i
    pltpu.sync_copy(acc_v, out_hbm.at[wid])


@jax.jit
def _quant_loss(ind2, eeg, ir, b):
    mesh = plsc.VectorSubcoreMesh(
        core_axis_name="c", subcore_axis_name="s",
        num_cores=_NC, num_subcores=_NS)
    partials = pl.kernel(
        _sc_body,
        out_type=jax.ShapeDtypeStruct((_NW, _LANES), jnp.float32),
        mesh=mesh,
        scratch_types=[
            pltpu.VMEM((_NCHUNK, _CHUNK), jnp.int32),
            pltpu.VMEM((_NBUF, _CHUNK, _DIM), jnp.float32),
            pltpu.VMEM((_NBUF, _CHUNK, _DIM), jnp.float32),
            pltpu.VMEM((_NBUF, _CHUNK, _DIM), jnp.float32),
            pltpu.VMEM((_LANES,), jnp.float32),
            pltpu.SemaphoreType.DMA((_NBUF, 3)),
        ],
    )(ind2, eeg, ir, b)
    return jnp.sum(partials)


def kernel(eeg, ir, ind, B, un_eeg, un_ir, device):
    ind2 = ind.astype(jnp.int32).reshape(_BATCH // _CHUNK, _CHUNK)
    return _quant_loss(ind2, eeg, ir, B)


# speedup vs baseline: 1.2772x; 1.0012x over previous
import functools

import jax
import jax.numpy as jnp
from jax import lax
from jax.experimental import pallas as pl
from jax.experimental.pallas import tpu as pltpu
from jax.experimental.pallas import tpu_sc as plsc

_NC = 2
_NS = 16
_NW = _NC * _NS
_LANES = 16
_BATCH = 16384
_DIM = 128
_BPW = _BATCH // _NW
_CHUNK = 64
_NCHUNK = _BPW // _CHUNK
_NBUF = 3
_VECS = _DIM // _LANES


def _sc_body(ind_hbm, eeg_hbm, ir_hbm, b_hbm, out_hbm,
             idx_v, rows_v, eeg_v, ir_v, acc_v, sems):
    c = lax.axis_index("c")
    s = lax.axis_index("s")
    wid = c * _NS + s
    base = wid * _BPW

    pltpu.sync_copy(ind_hbm.at[pl.ds(wid * 4, 4)], idx_v)

    def fire(ch):
        buf = ch % _NBUF
        row0 = base + ch * _CHUNK
        return (
            pltpu.async_copy(
                b_hbm.at[idx_v.at[ch // 2, pl.ds((ch % 2) * _CHUNK, _CHUNK)]],
                rows_v.at[buf], sems.at[buf, 0]),
            pltpu.async_copy(eeg_hbm.at[pl.ds(row0, _CHUNK)], eeg_v.at[buf],
                             sems.at[buf, 1]),
            pltpu.async_copy(ir_hbm.at[pl.ds(row0, _CHUNK)], ir_v.at[buf],
                             sems.at[buf, 2]),
        )

    zero = jnp.zeros((_LANES,), jnp.float32)
    acc_e = zero
    acc_i = zero

    inflight = [fire(0), fire(1)]
    for ch in range(_NCHUNK):
        buf = ch % _NBUF
        for cp in inflight[0]:
            cp.wait()
        inflight = inflight[1:]
        if ch + 2 < _NCHUNK:
            inflight.append(fire(ch + 2))

        @plsc.parallel_loop(0, _CHUNK, unroll=4, carry=(acc_e, acc_i))
        def _row(r, carry):
            a_e, a_i = carry
            for j in range(_VECS):
                col = j * _LANES
                b = rows_v[buf, r, pl.ds(col, _LANES)]
                e = eeg_v[buf, r, pl.ds(col, _LANES)]
                i = ir_v[buf, r, pl.ds(col, _LANES)]
                de = b - e
                di = b - i
                a_e = a_e + de * de
                a_i = a_i + di * di
            return a_e, a_i

        acc_e, acc_i = _row

    acc_v[...] = acc_e + acc_i
    pltpu.sync_copy(acc_v, out_hbm.at[wid])


@jax.jit
def _quant_loss(ind2, eeg, ir, b):
    mesh = plsc.VectorSubcoreMesh(
        core_axis_name="c", subcore_axis_name="s",
        num_cores=_NC, num_subcores=_NS)
    partials = pl.kernel(
        _sc_body,
        out_type=jax.ShapeDtypeStruct((_NW, _LANES), jnp.float32),
        mesh=mesh,
        scratch_types=[
            pltpu.VMEM((4, 128), jnp.int32),
            pltpu.VMEM((_NBUF, _CHUNK, _DIM), jnp.float32),
            pltpu.VMEM((_NBUF, _CHUNK, _DIM), jnp.float32),
            pltpu.VMEM((_NBUF, _CHUNK, _DIM), jnp.float32),
            pltpu.VMEM((_LANES,), jnp.float32),
            pltpu.SemaphoreType.DMA((_NBUF, 3)),
        ],
    )(ind2, eeg, ir, b)
    return jnp.sum(partials)


def kernel(eeg, ir, ind, B, un_eeg, un_ir, device):
    ind2 = ind.astype(jnp.int32).reshape(128, 128)
    return _quant_loss(ind2, eeg, ir, B)


# speedup vs baseline: 1.2865x; 1.0073x over previous
import functools

import jax
import jax.numpy as jnp
from jax import lax
from jax.experimental import pallas as pl
from jax.experimental.pallas import tpu as pltpu
from jax.experimental.pallas import tpu_sc as plsc

_NC = 2
_NS = 16
_NW = _NC * _NS
_LANES = 16
_BATCH = 16384
_DIM = 128
_BPW = _BATCH // _NW
_CHUNK = 64
_NCHUNK = _BPW // _CHUNK
_NBUF = 4
_VECS = _DIM // _LANES


def _sc_body(ind_hbm, eeg_hbm, ir_hbm, b_hbm, out_hbm,
             idx_v, rows_v, eeg_v, ir_v, acc_v, sems):
    c = lax.axis_index("c")
    s = lax.axis_index("s")
    wid = c * _NS + s
    base = wid * _BPW

    pltpu.sync_copy(ind_hbm.at[pl.ds(wid * 4, 4)], idx_v)

    def fire(ch):
        buf = ch % _NBUF
        row0 = base + ch * _CHUNK
        return (
            pltpu.async_copy(
                b_hbm.at[idx_v.at[ch // 2, pl.ds((ch % 2) * _CHUNK, _CHUNK)]],
                rows_v.at[buf], sems.at[buf, 0]),
            pltpu.async_copy(eeg_hbm.at[pl.ds(row0, _CHUNK)], eeg_v.at[buf],
                             sems.at[buf, 1]),
            pltpu.async_copy(ir_hbm.at[pl.ds(row0, _CHUNK)], ir_v.at[buf],
                             sems.at[buf, 2]),
        )

    zero = jnp.zeros((_LANES,), jnp.float32)
    acc_e = zero
    acc_i = zero

    inflight = [fire(0), fire(1), fire(2)]
    for ch in range(_NCHUNK):
        buf = ch % _NBUF
        for cp in inflight[0]:
            cp.wait()
        inflight = inflight[1:]
        if ch + 3 < _NCHUNK:
            inflight.append(fire(ch + 3))

        @plsc.parallel_loop(0, _CHUNK, unroll=4, carry=(acc_e, acc_i))
        def _row(r, carry):
            a_e, a_i = carry
            for j in range(_VECS):
                col = j * _LANES
                b = rows_v[buf, r, pl.ds(col, _LANES)]
                e = eeg_v[buf, r, pl.ds(col, _LANES)]
                i = ir_v[buf, r, pl.ds(col, _LANES)]
                de = b - e
                di = b - i
                a_e = a_e + de * de
                a_i = a_i + di * di
            return a_e, a_i

        acc_e, acc_i = _row

    acc_v[...] = acc_e + acc_i
    pltpu.sync_copy(acc_v, out_hbm.at[wid])


@jax.jit
def _quant_loss(ind2, eeg, ir, b):
    mesh = plsc.VectorSubcoreMesh(
        core_axis_name="c", subcore_axis_name="s",
        num_cores=_NC, num_subcores=_NS)
    partials = pl.kernel(
        _sc_body,
        out_type=jax.ShapeDtypeStruct((_NW, _LANES), jnp.float32),
        mesh=mesh,
        scratch_types=[
            pltpu.VMEM((4, 128), jnp.int32),
            pltpu.VMEM((_NBUF, _CHUNK, _DIM), jnp.float32),
            pltpu.VMEM((_NBUF, _CHUNK, _DIM), jnp.float32),
            pltpu.VMEM((_NBUF, _CHUNK, _DIM), jnp.float32),
            pltpu.VMEM((_LANES,), jnp.float32),
            pltpu.SemaphoreType.DMA((_NBUF, 3)),
        ],
    )(ind2, eeg, ir, b)
    return jnp.sum(partials)


def kernel(eeg, ir, ind, B, un_eeg, un_ir, device):
    ind2 = ind.astype(jnp.int32).reshape(128, 128)
    return _quant_loss(ind2, eeg, ir, B)
